# Initial kernel scaffold; baseline (speedup 1.0000x reference)
import jax
import jax.numpy as jnp
from jax.experimental import pallas as pl


def kernel(x, edge_index, edge_attr, batch, W1, a_src1, a_dst1, We1, a_e1, b1, W2, a_src2, a_dst2, We2, a_e2, b2, W3, a_src3, a_dst3, We3, a_e3, b3):
    raise NotImplementedError("write your pallas kernel here")


# speedup vs baseline: 1.0225x; 1.0225x over previous
import functools

import jax
import jax.numpy as jnp
from jax.experimental import pallas as pl

N = 10000; E = 320000; DIN = 128; DOUT = 128; HID = 16; H1 = 8; H3 = 1; EDIM = 16; G = 64


def _mm_body(x_ref, w_ref, o_ref):
    o_ref[...] = jnp.dot(x_ref[...], w_ref[...], preferred_element_type=jnp.float32)


def _pallas_mm(x, w, bm=1000):
    m, k = x.shape
    n = w.shape[1]
    return pl.pallas_call(
        _mm_body,
        grid=(m // bm,),
        in_specs=[pl.BlockSpec((bm, k), lambda i: (i, 0)),
                  pl.BlockSpec((k, n), lambda i: (0, 0))],
        out_specs=pl.BlockSpec((bm, n), lambda i: (i, 0)),
        out_shape=jax.ShapeDtypeStruct((m, n), jnp.float32),
    )(x, w)


def _add_self_loops(edge_index, edge_attr, n):
    dst = edge_index[1]
    ones = jnp.ones((edge_index.shape[1],), jnp.float32)
    deg = jax.ops.segment_sum(ones, dst, num_segments=n)
    loop_attr = jax.ops.segment_sum(edge_attr, dst, num_segments=n) / jnp.clip(deg, 1.0)[:, None]
    loops = jnp.arange(n, dtype=edge_index.dtype)
    ei = jnp.concatenate([edge_index, jnp.stack([loops, loops])], axis=1)
    ea = jnp.concatenate([edge_attr, loop_attr], axis=0)
    return ei, ea


def _gat_conv(x, edge_index, edge_attr, W, a_src, a_dst, We, a_e, b, heads, ch, concat, n):
    ei, ea = _add_self_loops(edge_index, edge_attr, n)
    src, dst = ei[0], ei[1]
    xp = _pallas_mm(x, W).reshape(n, heads, ch)
    alpha_src = jnp.sum(xp * a_src, axis=-1)
    alpha_dst = jnp.sum(xp * a_dst, axis=-1)
    ep = (ea @ We).reshape(-1, heads, ch)
    alpha_e = jnp.sum(ep * a_e, axis=-1)
    alpha = alpha_src[src] + alpha_dst[dst] + alpha_e
    alpha = jax.nn.leaky_relu(alpha, 0.2)
    amax = jax.ops.segment_max(alpha, dst, num_segments=n)
    amax = jnp.where(jnp.isfinite(amax), amax, 0.0)
    ex = jnp.exp(alpha - amax[dst])
    den = jax.ops.segment_sum(ex, dst, num_segments=n)
    alpha = ex / (den[dst] + 1e-16)
    out = jax.ops.segment_sum(xp[src] * alpha[:, :, None], dst, num_segments=n)
    out = out.reshape(n, heads * ch) if concat else jnp.mean(out, axis=1)
    return out + b


def kernel(x, edge_index, edge_attr, batch, W1, a_src1, a_dst1, We1, a_e1, b1, W2, a_src2, a_dst2, We2, a_e2, b2, W3, a_src3, a_dst3, We3, a_e3, b3):
    h = _gat_conv(x, edge_index, edge_attr, W1, a_src1, a_dst1, We1, a_e1, b1, H1, HID, True, N)
    h = jax.nn.elu(h)
    h = _gat_conv(h, edge_index, edge_attr, W2, a_src2, a_dst2, We2, a_e2, b2, H1, HID, True, N)
    h = jax.nn.elu(h)
    h = _gat_conv(h, edge_index, edge_attr, W3, a_src3, a_dst3, We3, a_e3, b3, H3, DOUT, False, N)
    cnt = jax.ops.segment_sum(jnp.ones((N,), jnp.float32), batch, num_segments=G)
    out = jax.ops.segment_sum(h, batch, num_segments=G) / jnp.clip(cnt, 1.0)[:, None]
    return out


# speedup vs baseline: 14.4137x; 14.0970x over previous
import functools

import jax
import jax.numpy as jnp
from jax import lax
from jax.experimental import pallas as pl
from jax.experimental.pallas import tpu as pltpu
from jax.experimental.pallas import tpu_sc as plsc

N = 10000; E = 320000; DIN = 128; DOUT = 128; HID = 16; H1 = 8; H3 = 1; EDIM = 16; G = 64

NP = 10240
CH = 128
EPAD = 323584
ACCW = 48
LW = 32
RPT = NP // 16

_mesh = functools.partial(
    plsc.VectorSubcoreMesh,
    core_axis_name="c", subcore_axis_name="s", num_cores=2, num_subcores=16)


def _sc_loop_attr(dst_p, ea_p):
    ept = EPAD // 32

    @functools.partial(
        pl.kernel,
        out_type=jax.ShapeDtypeStruct((2, NP, LW), jnp.float32),
        mesh=_mesh(),
        compiler_params=pltpu.CompilerParams(needs_layout_passes=False),
        scratch_types=[
            pltpu.VMEM_SHARED((NP, LW), jnp.float32),
            pltpu.VMEM((CH,), jnp.int32),
            pltpu.VMEM((CH, EDIM), jnp.float32),
            pltpu.VMEM((CH, LW), jnp.float32),
        ],
    )
    def k(dst_hbm, ea_hbm, out_hbm, acc_sh, dst_v, ea_v, pay):
        c = lax.axis_index("c")
        s = lax.axis_index("s")

        zv = jnp.zeros((16,), jnp.float32)

        def zrow(j, _):
            for t in range(LW // 16):
                pay[j, pl.ds(t * 16, 16)] = zv
            return 0
        lax.fori_loop(0, CH, zrow, 0)
        for r in range(RPT // CH):
            pltpu.sync_copy(pay, acc_sh.at[pl.ds(s * RPT + r * CH, CH)])
        plsc.subcore_barrier()

        onehot = jnp.where(lax.iota(jnp.int32, 16) == 0, 1.0, 0.0).astype(jnp.float32)

        def orow(j, _):
            pay[j, pl.ds(16, 16)] = onehot
            return 0
        lax.fori_loop(0, CH, orow, 0)

        w = c * 16 + s

        def chunk(i, _):
            off = w * ept + i * CH
            pltpu.sync_copy(dst_hbm.at[pl.ds(off, CH)], dst_v)
            pltpu.sync_copy(ea_hbm.at[pl.ds(off, CH)], ea_v)

            def ebody(j, _):
                pay[j, pl.ds(0, 16)] = ea_v[j, pl.ds(0, 16)]
                return 0
            lax.fori_loop(0, CH, ebody, 0)
            pltpu.sync_copy(pay, acc_sh.at[dst_v], add=True)
            return 0
        lax.fori_loop(0, ept // CH, chunk, 0)
        plsc.subcore_barrier()
        for r in range(RPT // CH):
            sl = pl.ds(s * RPT + r * CH, CH)
            pltpu.sync_copy(acc_sh.at[sl], out_hbm.at[c].at[sl])

    return k(dst_p, ea_p)


def _sc_edge(src_p, dst_p, ae, asrc_t, adst_t, xp_slab):
    ept = EPAD // 8
    art = 2 * NP // 16

    @functools.partial(
        pl.kernel,
        out_type=jax.ShapeDtypeStruct((2, 2 * NP, ACCW), jnp.float32),
        mesh=_mesh(),
        compiler_params=pltpu.CompilerParams(
            needs_layout_passes=False, use_tc_tiling_on_sc=False),
        scratch_types=[
            pltpu.VMEM_SHARED((2 * NP, ACCW), jnp.float32),
            pltpu.VMEM((NP * 2,), jnp.float32),
            pltpu.VMEM((NP * 2,), jnp.float32),
            pltpu.VMEM((CH,), jnp.int32),
            pltpu.VMEM((CH,), jnp.int32),
            pltpu.VMEM((CH,), jnp.int32),
            pltpu.VMEM((CH * 2,), jnp.float32),
            pltpu.VMEM((2, CH), jnp.float32),
            pltpu.VMEM((CH, 32), jnp.float32),
            pltpu.VMEM((CH, ACCW), jnp.float32),
            pltpu.SemaphoreType.DMA,
        ],
    )
    def k(src_hbm, dst_hbm, ae_hbm, asrc_hbm, adst_hbm, xp_hbm, out_hbm,
          acc_sh, asrc_l, adst_l, src_v, dst_v, xoff_v, ae_v, ex_v, xbuf, pay,
          sem):
        c = lax.axis_index("c")
        s = lax.axis_index("s")
        hp = lax.rem(s, 2)
        octant = lax.div(s, 2)
        q = c * 2 + hp
        pltpu.sync_copy(asrc_hbm.at[q], asrc_l)
        pltpu.sync_copy(adst_hbm.at[q], adst_l)

        zv = jnp.zeros((16,), jnp.float32)

        def zrow(j, _):
            for t in range(ACCW // 16):
                pay[j, pl.ds(t * 16, 16)] = zv
            return 0
        lax.fori_loop(0, CH, zrow, 0)
        for r in range(art // CH):
            pltpu.sync_copy(pay, acc_sh.at[pl.ds(s * art + r * CH, CH)])
        plsc.subcore_barrier()

        iota16 = lax.iota(jnp.int32, 16)
        hots = [jnp.where(iota16 == h, 1.0, 0.0).astype(jnp.float32)
                for h in range(2)]

        def chunk(i, _):
            off = octant * ept + i * CH
            pltpu.sync_copy(src_hbm.at[pl.ds(off, CH)], src_v)
            pltpu.sync_copy(dst_hbm.at[pl.ds(off, CH)], dst_v)
            pltpu.sync_copy(ae_hbm.at[q].at[pl.ds(off * 2, CH * 2)], ae_v)
            for g in range(CH // 16):
                sl = pl.ds(g * 16, 16)
                sidx = src_v[sl]
                didx = dst_v[sl]
                sidx2 = sidx * 2
                didx2 = didx * 2
                rows2 = (iota16 + g * 16) * 2
                for hh in range(2):
                    a = (plsc.load_gather(asrc_l, [sidx2 + hh])
                         + plsc.load_gather(adst_l, [didx2 + hh])
                         + plsc.load_gather(ae_v, [rows2 + hh]))
                    a = jnp.maximum(a, 0.2 * a)
                    ex_v[hh, sl] = jnp.exp(a)
                xoff_v[sl] = sidx + q * NP
                dst_v[sl] = didx + hp * NP
            pltpu.async_copy(xp_hbm.at[xoff_v], xbuf, sem).wait()

            def gbody(g, _):
                ev0 = ex_v[0, pl.ds(g * 16, 16)]
                ev1 = ex_v[1, pl.ds(g * 16, 16)]
                for jj in range(16):
                    row = g * 16 + jj
                    e0 = ev0[jj]
                    e1 = ev1[jj]
                    pay[row, pl.ds(0, 16)] = e0 * xbuf[row, pl.ds(0, 16)]
                    pay[row, pl.ds(16, 16)] = e1 * xbuf[row, pl.ds(16, 16)]
                    pay[row, pl.ds(32, 16)] = e0 * hots[0] + e1 * hots[1]
                return 0
            lax.fori_loop(0, CH // 16, gbody, 0)
            pltpu.sync_copy(pay, acc_sh.at[dst_v], add=True)
            return 0
        lax.fori_loop(0, ept // CH, chunk, 0)
        plsc.subcore_barrier()
        for r in range(art // CH):
            sl = pl.ds(s * art + r * CH, CH)
            pltpu.sync_copy(acc_sh.at[sl], out_hbm.at[c].at[sl])

    return k(src_p, dst_p, ae, asrc_t, adst_t, xp_slab)


def _tc_loop_finish(acc0):
    def body(a_ref, o_ref):
        ea = a_ref[0, :, :16] + a_ref[1, :, :16]
        cnt = a_ref[0, :, 16:17] + a_ref[1, :, 16:17]
        o_ref[...] = ea / jnp.maximum(cnt, 1.0)

    return pl.pallas_call(
        body,
        grid=(10,),
        in_specs=[pl.BlockSpec((2, NP // 10, LW), lambda i: (0, i, 0))],
        out_specs=pl.BlockSpec((NP // 10, EDIM), lambda i: (i, 0)),
        out_shape=jax.ShapeDtypeStruct((NP, EDIM), jnp.float32),
    )(acc0)


def _tc_edge_prep(ea_p, we1, me1, we2, me2, we3, me3):
    def body(ea_ref, w1_ref, m1_ref, w2_ref, m2_ref, w3_ref, m3_ref,
             o1_ref, o2_ref, o3_ref):
        ea = ea_ref[...]
        for w_ref, m_ref, o_ref in ((w1_ref, m1_ref, o1_ref),
                                    (w2_ref, m2_ref, o2_ref),
                                    (w3_ref, m3_ref, o3_ref)):
            wm = jnp.dot(w_ref[...], m_ref[...], preferred_element_type=jnp.float32)
            o_ref[...] = jnp.dot(ea, wm, preferred_element_type=jnp.float32)

    bm = 4096
    wspec = pl.BlockSpec((EDIM, 128), lambda i: (0, 0))
    mspec = pl.BlockSpec((128, 8), lambda i: (0, 0))
    ospec = pl.BlockSpec((bm, 8), lambda i: (i, 0))
    outs = pl.pallas_call(
        body,
        grid=(EPAD // bm,),
        in_specs=[pl.BlockSpec((bm, EDIM), lambda i: (i, 0)),
                  wspec, mspec, wspec, mspec, wspec, mspec],
        out_specs=[ospec, ospec, ospec],
        out_shape=[jax.ShapeDtypeStruct((EPAD, 8), jnp.float32)] * 3,
    )(ea_p, we1, me1, we2, me2, we3, me3)
    return outs


def _tc_prep(h, w, msrc, mdst, la, we, me):
    def body(h_ref, w_ref, ms_ref, md_ref, la_ref, we_ref, me_ref,
             xp_ref, as_ref, ad_ref, exl_ref):
        xp = jnp.dot(h_ref[...], w_ref[...], preferred_element_type=jnp.float32)
        asrc = jnp.dot(xp, ms_ref[...], preferred_element_type=jnp.float32)
        adst = jnp.dot(xp, md_ref[...], preferred_element_type=jnp.float32)
        wm = jnp.dot(we_ref[...], me_ref[...], preferred_element_type=jnp.float32)
        aeloop = jnp.dot(la_ref[...], wm, preferred_element_type=jnp.float32)
        al = asrc + adst + aeloop
        al = jnp.maximum(al, 0.2 * al)
        exl_ref[...] = jnp.exp(al)
        xp_ref[...] = xp
        as_ref[...] = asrc
        ad_ref[...] = adst

    bm = NP // 10
    return pl.pallas_call(
        body,
        grid=(10,),
        in_specs=[pl.BlockSpec((bm, 128), lambda i: (i, 0)),
                  pl.BlockSpec((128, 128), lambda i: (0, 0)),
                  pl.BlockSpec((128, 8), lambda i: (0, 0)),
                  pl.BlockSpec((128, 8), lambda i: (0, 0)),
                  pl.BlockSpec((bm, EDIM), lambda i: (i, 0)),
                  pl.BlockSpec((EDIM, 128), lambda i: (0, 0)),
                  pl.BlockSpec((128, 8), lambda i: (0, 0))],
        out_specs=[pl.BlockSpec((bm, 128), lambda i: (i, 0)),
                   pl.BlockSpec((bm, 8), lambda i: (i, 0)),
                   pl.BlockSpec((bm, 8), lambda i: (i, 0)),
                   pl.BlockSpec((bm, 8), lambda i: (i, 0))],
        out_shape=[jax.ShapeDtypeStruct((NP, 128), jnp.float32),
                   jax.ShapeDtypeStruct((NP, 8), jnp.float32),
                   jax.ShapeDtypeStruct((NP, 8), jnp.float32),
                   jax.ShapeDtypeStruct((NP, 8), jnp.float32)],
    )(h, w, msrc, mdst, la, we, me)


def _tc_norm(acc, exl, xp_sc, b, k4, do_elu):
    def body(acc_ref, exl_ref, xp_ref, b_ref, k4_ref, o_ref):
        k4 = k4_ref[...]
        for sc in range(2):
            exl_sc = exl_ref[:, sc * 4:(sc + 1) * 4]
            e64 = jnp.dot(exl_sc, k4, preferred_element_type=jnp.float32)
            xp = xp_ref[:, sc * 64:(sc + 1) * 64]
            num = jnp.concatenate([acc_ref[sc, 0, :, :32],
                                   acc_ref[sc, 1, :, :32]], axis=1)
            num = num + e64 * xp
            den4 = jnp.concatenate([acc_ref[sc, 0, :, 32:34],
                                    acc_ref[sc, 1, :, 32:34]], axis=1) + exl_sc
            den = jnp.dot(den4, k4, preferred_element_type=jnp.float32)
            o = num / den + b_ref[0, sc * 64:(sc + 1) * 64]
            if do_elu:
                o = jnp.where(o > 0, o, jnp.exp(jnp.minimum(o, 0.0)) - 1.0)
            o_ref[:, sc * 64:(sc + 1) * 64] = o

    bm = NP // 10
    return pl.pallas_call(
        body,
        grid=(10,),
        in_specs=[pl.BlockSpec((2, 2, bm, ACCW), lambda i: (0, 0, i, 0)),
                  pl.BlockSpec((bm, 8), lambda i: (i, 0)),
                  pl.BlockSpec((bm, 128), lambda i: (i, 0)),
                  pl.BlockSpec((1, 128), lambda i: (0, 0)),
                  pl.BlockSpec((4, 64), lambda i: (0, 0))],
        out_specs=pl.BlockSpec((bm, 128), lambda i: (i, 0)),
        out_shape=jax.ShapeDtypeStruct((NP, 128), jnp.float32),
    )(acc, exl, xp_sc, b, k4)


def _tc_pool(h3, p):
    def body(p_ref, h_ref, o_ref):
        pm = p_ref[...]
        s = lax.dot_general(pm, h_ref[...], (((0,), (0,)), ((), ())),
                            preferred_element_type=jnp.float32)
        cnt = jnp.sum(pm, axis=0)[:, None]
        o_ref[...] = s / jnp.maximum(cnt, 1.0)

    return pl.pallas_call(
        body,
        in_specs=[pl.BlockSpec((NP, G), lambda: (0, 0)),
                  pl.BlockSpec((NP, 128), lambda: (0, 0))],
        out_specs=pl.BlockSpec((G, 128), lambda: (0, 0)),
        out_shape=jax.ShapeDtypeStruct((G, 128), jnp.float32),
    )(p, h3)


def _mask8(a):
    return (jnp.eye(8, dtype=jnp.float32)[:, None, :] * a[0][:, :, None]).reshape(128, 8)


def _mask1(a):
    return jnp.tile(a[0, 0][:, None], (1, 8))


def kernel(x, edge_index, edge_attr, batch, W1, a_src1, a_dst1, We1, a_e1, b1,
           W2, a_src2, a_dst2, We2, a_e2, b2, W3, a_src3, a_dst3, We3, a_e3, b3):
    f32 = jnp.float32
    src = edge_index[0]
    dst = edge_index[1]
    epad = EPAD - E
    src_p = jnp.concatenate([src, jnp.full((epad,), N, jnp.int32)])
    dst_p = jnp.concatenate([dst, jnp.full((epad,), N, jnp.int32)])
    ea_p = jnp.concatenate([edge_attr, jnp.zeros((epad, EDIM), f32)])
    x_p = jnp.concatenate([x, jnp.zeros((NP - N, DIN), f32)])

    p = (batch[:, None] == jnp.arange(G, dtype=jnp.int32)[None, :]).astype(f32)
    p = jnp.concatenate([p, jnp.zeros((NP - N, G), f32)])

    k4 = (jnp.eye(4, dtype=f32)[:, :, None] * jnp.ones((1, 1, 16), f32)).reshape(4, 64)

    msrc = (_mask8(a_src1), _mask8(a_src2), _mask1(a_src3))
    mdst = (_mask8(a_dst1), _mask8(a_dst2), _mask1(a_dst3))
    me = (_mask8(a_e1), _mask8(a_e2), _mask1(a_e3))
    ws = (W1, W2, W3)
    wes = (We1, We2, We3)
    bs = (b1.reshape(1, 128), b2.reshape(1, 128), b3.reshape(1, 128))

    acc0 = _sc_loop_attr(dst_p, ea_p)
    la = _tc_loop_finish(acc0)
    aes = _tc_edge_prep(ea_p, We1, me[0], We2, me[1], We3, me[2])

    h = x_p
    for l in range(3):
        xp, asrc8, adst8, exl = _tc_prep(h, ws[l], msrc[l], mdst[l], la,
                                         wes[l], me[l])
        ae_t = aes[l].reshape(EPAD, 4, 2).transpose(1, 0, 2).reshape(4, EPAD * 2)
        asrc_t = asrc8.reshape(NP, 4, 2).transpose(1, 0, 2).reshape(4, NP * 2)
        adst_t = adst8.reshape(NP, 4, 2).transpose(1, 0, 2).reshape(4, NP * 2)
        xp_t = xp.reshape(NP, 4, 32).transpose(1, 0, 2).reshape(4 * NP, 32)
        acc = _sc_edge(src_p, dst_p, ae_t, asrc_t, adst_t, xp_t)
        h = _tc_norm(acc.reshape(2, 2, NP, ACCW), exl, xp, bs[l], k4,
                     do_elu=(l < 2))

    out = _tc_pool(h, p)
    return out


# speedup vs baseline: 20.7557x; 1.4400x over previous
import functools

import jax
import jax.numpy as jnp
from jax import lax
from jax.experimental import pallas as pl
from jax.experimental.pallas import tpu as pltpu
from jax.experimental.pallas import tpu_sc as plsc

N = 10000; E = 320000; DIN = 128; DOUT = 128; HID = 16; H1 = 8; H3 = 1; EDIM = 16; G = 64

NP = 10240
CH = 128
EPAD = 323584
ACCW = 48
LW = 32
RPT = NP // 16

_mesh = functools.partial(
    plsc.VectorSubcoreMesh,
    core_axis_name="c", subcore_axis_name="s", num_cores=2, num_subcores=16)


def _sc_loop_attr(dst_p, ea_p):
    ept = EPAD // 32

    @functools.partial(
        pl.kernel,
        out_type=jax.ShapeDtypeStruct((2, NP, LW), jnp.float32),
        mesh=_mesh(),
        compiler_params=pltpu.CompilerParams(needs_layout_passes=False),
        scratch_types=[
            pltpu.VMEM_SHARED((NP, LW), jnp.float32),
            pltpu.VMEM((CH,), jnp.int32),
            pltpu.VMEM((CH, EDIM), jnp.float32),
            pltpu.VMEM((CH, LW), jnp.float32),
        ],
    )
    def k(dst_hbm, ea_hbm, out_hbm, acc_sh, dst_v, ea_v, pay):
        c = lax.axis_index("c")
        s = lax.axis_index("s")

        zv = jnp.zeros((16,), jnp.float32)

        def zrow(j, _):
            for t in range(LW // 16):
                pay[j, pl.ds(t * 16, 16)] = zv
            return 0
        lax.fori_loop(0, CH, zrow, 0)
        for r in range(RPT // CH):
            pltpu.sync_copy(pay, acc_sh.at[pl.ds(s * RPT + r * CH, CH)])
        plsc.subcore_barrier()

        onehot = jnp.where(lax.iota(jnp.int32, 16) == 0, 1.0, 0.0).astype(jnp.float32)

        def orow(j, _):
            pay[j, pl.ds(16, 16)] = onehot
            return 0
        lax.fori_loop(0, CH, orow, 0)

        w = c * 16 + s

        def chunk(i, _):
            off = w * ept + i * CH
            pltpu.sync_copy(dst_hbm.at[pl.ds(off, CH)], dst_v)
            pltpu.sync_copy(ea_hbm.at[pl.ds(off, CH)], ea_v)

            def ebody(j, _):
                pay[j, pl.ds(0, 16)] = ea_v[j, pl.ds(0, 16)]
                return 0
            lax.fori_loop(0, CH, ebody, 0)
            pltpu.sync_copy(pay, acc_sh.at[dst_v], add=True)
            return 0
        lax.fori_loop(0, ept // CH, chunk, 0)
        plsc.subcore_barrier()
        for r in range(RPT // CH):
            sl = pl.ds(s * RPT + r * CH, CH)
            pltpu.sync_copy(acc_sh.at[sl], out_hbm.at[c].at[sl])

    return k(dst_p, ea_p)


def _sc_edge(src_p, dst_p, ae, asrc_t, adst_t, xp_slab):
    ept = EPAD // 8
    art = 2 * NP // 16

    @functools.partial(
        pl.kernel,
        out_type=jax.ShapeDtypeStruct((2, 2 * NP, ACCW), jnp.float32),
        mesh=_mesh(),
        compiler_params=pltpu.CompilerParams(
            needs_layout_passes=False, use_tc_tiling_on_sc=False),
        scratch_types=[
            pltpu.VMEM_SHARED((2 * NP, ACCW), jnp.float32),
            pltpu.VMEM((NP * 2,), jnp.float32),
            pltpu.VMEM((NP * 2,), jnp.float32),
            pltpu.VMEM((CH,), jnp.int32),
            pltpu.VMEM((CH,), jnp.int32),
            pltpu.VMEM((CH,), jnp.int32),
            pltpu.VMEM((2, CH), jnp.float32),
            pltpu.VMEM((2, CH), jnp.float32),
            pltpu.VMEM((CH, 32), jnp.float32),
            pltpu.VMEM((CH, ACCW), jnp.float32),
            pltpu.SemaphoreType.DMA,
        ],
    )
    def k(src_hbm, dst_hbm, ae_hbm, asrc_hbm, adst_hbm, xp_hbm, out_hbm,
          acc_sh, asrc_l, adst_l, src_v, dst_v, xoff_v, ae_v, ex_v, xbuf, pay,
          sem):
        c = lax.axis_index("c")
        s = lax.axis_index("s")
        hp = lax.rem(s, 2)
        octant = lax.div(s, 2)
        q = c * 2 + hp
        h0 = q * 2
        pltpu.sync_copy(asrc_hbm.at[h0], asrc_l.at[pl.ds(0, NP)])
        pltpu.sync_copy(asrc_hbm.at[h0 + 1], asrc_l.at[pl.ds(NP, NP)])
        pltpu.sync_copy(adst_hbm.at[h0], adst_l.at[pl.ds(0, NP)])
        pltpu.sync_copy(adst_hbm.at[h0 + 1], adst_l.at[pl.ds(NP, NP)])

        zv = jnp.zeros((16,), jnp.float32)

        def zrow(j, _):
            for t in range(ACCW // 16):
                pay[j, pl.ds(t * 16, 16)] = zv
            return 0
        lax.fori_loop(0, CH, zrow, 0)
        for r in range(art // CH):
            pltpu.sync_copy(pay, acc_sh.at[pl.ds(s * art + r * CH, CH)])
        plsc.subcore_barrier()

        iota16 = lax.iota(jnp.int32, 16)
        hots = [jnp.where(iota16 == h, 1.0, 0.0).astype(jnp.float32)
                for h in range(2)]

        def chunk(i, _):
            off = octant * ept + i * CH
            pltpu.sync_copy(src_hbm.at[pl.ds(off, CH)], src_v)
            pltpu.sync_copy(dst_hbm.at[pl.ds(off, CH)], dst_v)
            pltpu.sync_copy(ae_hbm.at[h0].at[pl.ds(off, CH)], ae_v.at[0])
            pltpu.sync_copy(ae_hbm.at[h0 + 1].at[pl.ds(off, CH)], ae_v.at[1])
            for g in range(CH // 16):
                sl = pl.ds(g * 16, 16)
                sidx = src_v[sl]
                didx = dst_v[sl]
                for hh in range(2):
                    a = (plsc.load_gather(asrc_l, [sidx + hh * NP])
                         + plsc.load_gather(adst_l, [didx + hh * NP])
                         + ae_v[hh, sl])
                    a = jnp.maximum(a, 0.2 * a)
                    ex_v[hh, sl] = jnp.exp(a)
                xoff_v[sl] = sidx + q * NP
                dst_v[sl] = didx + hp * NP
            pltpu.async_copy(xp_hbm.at[xoff_v], xbuf, sem).wait()

            def gbody(g, _):
                ev0 = ex_v[0, pl.ds(g * 16, 16)]
                ev1 = ex_v[1, pl.ds(g * 16, 16)]
                for jj in range(16):
                    row = g * 16 + jj
                    e0 = ev0[jj]
                    e1 = ev1[jj]
                    pay[row, pl.ds(0, 16)] = e0 * xbuf[row, pl.ds(0, 16)]
                    pay[row, pl.ds(16, 16)] = e1 * xbuf[row, pl.ds(16, 16)]
                    pay[row, pl.ds(32, 16)] = e0 * hots[0] + e1 * hots[1]
                return 0
            lax.fori_loop(0, CH // 16, gbody, 0)
            pltpu.sync_copy(pay, acc_sh.at[dst_v], add=True)
            return 0
        lax.fori_loop(0, ept // CH, chunk, 0)
        plsc.subcore_barrier()
        for r in range(art // CH):
            sl = pl.ds(s * art + r * CH, CH)
            pltpu.sync_copy(acc_sh.at[sl], out_hbm.at[c].at[sl])

    return k(src_p, dst_p, ae, asrc_t, adst_t, xp_slab)


def _tc_loop_finish(acc0):
    def body(a_ref, o_ref):
        ea = a_ref[0, :, :16] + a_ref[1, :, :16]
        cnt = a_ref[0, :, 16:17] + a_ref[1, :, 16:17]
        o_ref[...] = ea / jnp.maximum(cnt, 1.0)

    return pl.pallas_call(
        body,
        grid=(10,),
        in_specs=[pl.BlockSpec((2, NP // 10, LW), lambda i: (0, i, 0))],
        out_specs=pl.BlockSpec((NP // 10, EDIM), lambda i: (i, 0)),
        out_shape=jax.ShapeDtypeStruct((NP, EDIM), jnp.float32),
    )(acc0)


def _tc_edge_prep(ea_p, we1, me1, we2, me2, we3, me3):
    def body(ea_ref, w1_ref, m1_ref, w2_ref, m2_ref, w3_ref, m3_ref,
             o1_ref, o2_ref, o3_ref):
        ea = ea_ref[...]
        for w_ref, m_ref, o_ref in ((w1_ref, m1_ref, o1_ref),
                                    (w2_ref, m2_ref, o2_ref),
                                    (w3_ref, m3_ref, o3_ref)):
            wm = jnp.dot(w_ref[...], m_ref[...], preferred_element_type=jnp.float32)
            o_ref[...] = lax.dot_general(
                wm, ea, (((0,), (1,)), ((), ())),
                preferred_element_type=jnp.float32)

    bm = 4096
    wspec = pl.BlockSpec((EDIM, 128), lambda i: (0, 0))
    mspec = pl.BlockSpec((128, 8), lambda i: (0, 0))
    ospec = pl.BlockSpec((8, bm), lambda i: (0, i))
    outs = pl.pallas_call(
        body,
        grid=(EPAD // bm,),
        in_specs=[pl.BlockSpec((bm, EDIM), lambda i: (i, 0)),
                  wspec, mspec, wspec, mspec, wspec, mspec],
        out_specs=[ospec, ospec, ospec],
        out_shape=[jax.ShapeDtypeStruct((8, EPAD), jnp.float32)] * 3,
    )(ea_p, we1, me1, we2, me2, we3, me3)
    return outs


def _tc_prep(h, w, msrc, mdst, la, we, me):
    def body(h_ref, w_ref, ms_ref, md_ref, la_ref, we_ref, me_ref,
             xp_ref, xps_ref, as_ref, ad_ref, exl_ref):
        xp = jnp.dot(h_ref[...], w_ref[...], preferred_element_type=jnp.float32)
        asrc = jnp.dot(xp, ms_ref[...], preferred_element_type=jnp.float32)
        adst = jnp.dot(xp, md_ref[...], preferred_element_type=jnp.float32)
        wm = jnp.dot(we_ref[...], me_ref[...], preferred_element_type=jnp.float32)
        aeloop = jnp.dot(la_ref[...], wm, preferred_element_type=jnp.float32)
        al = asrc + adst + aeloop
        al = jnp.maximum(al, 0.2 * al)
        exl_ref[...] = jnp.exp(al)
        xp_ref[...] = xp
        for qq in range(4):
            xps_ref[qq] = xp[:, qq * 32:(qq + 1) * 32]
        as_ref[...] = lax.dot_general(
            ms_ref[...], xp, (((0,), (1,)), ((), ())),
            preferred_element_type=jnp.float32)
        ad_ref[...] = lax.dot_general(
            md_ref[...], xp, (((0,), (1,)), ((), ())),
            preferred_element_type=jnp.float32)

    bm = NP // 10
    return pl.pallas_call(
        body,
        grid=(10,),
        in_specs=[pl.BlockSpec((bm, 128), lambda i: (i, 0)),
                  pl.BlockSpec((128, 128), lambda i: (0, 0)),
                  pl.BlockSpec((128, 8), lambda i: (0, 0)),
                  pl.BlockSpec((128, 8), lambda i: (0, 0)),
                  pl.BlockSpec((bm, EDIM), lambda i: (i, 0)),
                  pl.BlockSpec((EDIM, 128), lambda i: (0, 0)),
                  pl.BlockSpec((128, 8), lambda i: (0, 0))],
        out_specs=[pl.BlockSpec((bm, 128), lambda i: (i, 0)),
                   pl.BlockSpec((4, bm, 32), lambda i: (0, i, 0)),
                   pl.BlockSpec((8, bm), lambda i: (0, i)),
                   pl.BlockSpec((8, bm), lambda i: (0, i)),
                   pl.BlockSpec((bm, 8), lambda i: (i, 0))],
        out_shape=[jax.ShapeDtypeStruct((NP, 128), jnp.float32),
                   jax.ShapeDtypeStruct((4, NP, 32), jnp.float32),
                   jax.ShapeDtypeStruct((8, NP), jnp.float32),
                   jax.ShapeDtypeStruct((8, NP), jnp.float32),
                   jax.ShapeDtypeStruct((NP, 8), jnp.float32)],
    )(h, w, msrc, mdst, la, we, me)


def _tc_norm(acc, exl, xp_sc, b, k4, do_elu):
    def body(acc_ref, exl_ref, xp_ref, b_ref, k4_ref, o_ref):
        k4 = k4_ref[...]
        for sc in range(2):
            exl_sc = exl_ref[:, sc * 4:(sc + 1) * 4]
            e64 = jnp.dot(exl_sc, k4, preferred_element_type=jnp.float32)
            xp = xp_ref[:, sc * 64:(sc + 1) * 64]
            num = jnp.concatenate([acc_ref[sc, 0, :, :32],
                                   acc_ref[sc, 1, :, :32]], axis=1)
            num = num + e64 * xp
            den4 = jnp.concatenate([acc_ref[sc, 0, :, 32:34],
                                    acc_ref[sc, 1, :, 32:34]], axis=1) + exl_sc
            den = jnp.dot(den4, k4, preferred_element_type=jnp.float32)
            o = num / den + b_ref[0, sc * 64:(sc + 1) * 64]
            if do_elu:
                o = jnp.where(o > 0, o, jnp.exp(jnp.minimum(o, 0.0)) - 1.0)
            o_ref[:, sc * 64:(sc + 1) * 64] = o

    bm = NP // 10
    return pl.pallas_call(
        body,
        grid=(10,),
        in_specs=[pl.BlockSpec((2, 2, bm, ACCW), lambda i: (0, 0, i, 0)),
                  pl.BlockSpec((bm, 8), lambda i: (i, 0)),
                  pl.BlockSpec((bm, 128), lambda i: (i, 0)),
                  pl.BlockSpec((1, 128), lambda i: (0, 0)),
                  pl.BlockSpec((4, 64), lambda i: (0, 0))],
        out_specs=pl.BlockSpec((bm, 128), lambda i: (i, 0)),
        out_shape=jax.ShapeDtypeStruct((NP, 128), jnp.float32),
    )(acc, exl, xp_sc, b, k4)


def _tc_pool(h3, p):
    def body(p_ref, h_ref, o_ref):
        pm = p_ref[...]
        s = lax.dot_general(pm, h_ref[...], (((0,), (0,)), ((), ())),
                            preferred_element_type=jnp.float32)
        cnt = jnp.sum(pm, axis=0)[:, None]
        o_ref[...] = s / jnp.maximum(cnt, 1.0)

    return pl.pallas_call(
        body,
        in_specs=[pl.BlockSpec((NP, G), lambda: (0, 0)),
                  pl.BlockSpec((NP, 128), lambda: (0, 0))],
        out_specs=pl.BlockSpec((G, 128), lambda: (0, 0)),
        out_shape=jax.ShapeDtypeStruct((G, 128), jnp.float32),
    )(p, h3)


def _mask8(a):
    return (jnp.eye(8, dtype=jnp.float32)[:, None, :] * a[0][:, :, None]).reshape(128, 8)


def _mask1(a):
    return jnp.tile(a[0, 0][:, None], (1, 8))


def kernel(x, edge_index, edge_attr, batch, W1, a_src1, a_dst1, We1, a_e1, b1,
           W2, a_src2, a_dst2, We2, a_e2, b2, W3, a_src3, a_dst3, We3, a_e3, b3):
    f32 = jnp.float32
    src = edge_index[0]
    dst = edge_index[1]
    epad = EPAD - E
    src_p = jnp.concatenate([src, jnp.full((epad,), N, jnp.int32)])
    dst_p = jnp.concatenate([dst, jnp.full((epad,), N, jnp.int32)])
    ea_p = jnp.concatenate([edge_attr, jnp.zeros((epad, EDIM), f32)])
    x_p = jnp.concatenate([x, jnp.zeros((NP - N, DIN), f32)])

    p = (batch[:, None] == jnp.arange(G, dtype=jnp.int32)[None, :]).astype(f32)
    p = jnp.concatenate([p, jnp.zeros((NP - N, G), f32)])

    k4 = (jnp.eye(4, dtype=f32)[:, :, None] * jnp.ones((1, 1, 16), f32)).reshape(4, 64)

    msrc = (_mask8(a_src1), _mask8(a_src2), _mask1(a_src3))
    mdst = (_mask8(a_dst1), _mask8(a_dst2), _mask1(a_dst3))
    me = (_mask8(a_e1), _mask8(a_e2), _mask1(a_e3))
    ws = (W1, W2, W3)
    wes = (We1, We2, We3)
    bs = (b1.reshape(1, 128), b2.reshape(1, 128), b3.reshape(1, 128))

    acc0 = _sc_loop_attr(dst_p, ea_p)
    la = _tc_loop_finish(acc0)
    aes = _tc_edge_prep(ea_p, We1, me[0], We2, me[1], We3, me[2])

    h = x_p
    for l in range(3):
        xp, xps, asrc_t, adst_t, exl = _tc_prep(h, ws[l], msrc[l], mdst[l], la,
                                                wes[l], me[l])
        acc = _sc_edge(src_p, dst_p, aes[l], asrc_t, adst_t,
                       xps.reshape(4 * NP, 32))
        h = _tc_norm(acc.reshape(2, 2, NP, ACCW), exl, xp, bs[l], k4,
                     do_elu=(l < 2))

    out = _tc_pool(h, p)
    return out


# speedup vs baseline: 38.8039x; 1.8696x over previous
import functools

import jax
import jax.numpy as jnp
from jax import lax
from jax.experimental import pallas as pl
from jax.experimental.pallas import tpu as pltpu
from jax.experimental.pallas import tpu_sc as plsc

N = 10000; E = 320000; DIN = 128; DOUT = 128; HID = 16; H1 = 8; H3 = 1; EDIM = 16; G = 64

NP = 10240
CH = 128
EPAD = 323584
ACCW = 48
LW = 32
RPT = NP // 16

_mesh = functools.partial(
    plsc.VectorSubcoreMesh,
    core_axis_name="c", subcore_axis_name="s", num_cores=2, num_subcores=16)


def _sc_loop_attr(dst_p, ea_p):
    ept = EPAD // 32

    @functools.partial(
        pl.kernel,
        out_type=jax.ShapeDtypeStruct((2, NP, LW), jnp.float32),
        mesh=_mesh(),
        compiler_params=pltpu.CompilerParams(needs_layout_passes=False),
        scratch_types=[
            pltpu.VMEM_SHARED((NP, LW), jnp.float32),
            pltpu.VMEM((CH,), jnp.int32),
            pltpu.VMEM((CH, EDIM), jnp.float32),
            pltpu.VMEM((CH, LW), jnp.float32),
        ],
    )
    def k(dst_hbm, ea_hbm, out_hbm, acc_sh, dst_v, ea_v, pay):
        c = lax.axis_index("c")
        s = lax.axis_index("s")

        zv = jnp.zeros((16,), jnp.float32)

        def zrow(j, _):
            for t in range(LW // 16):
                pay[j, pl.ds(t * 16, 16)] = zv
            return 0
        lax.fori_loop(0, CH, zrow, 0)
        for r in range(RPT // CH):
            pltpu.sync_copy(pay, acc_sh.at[pl.ds(s * RPT + r * CH, CH)])
        plsc.subcore_barrier()

        onehot = jnp.where(lax.iota(jnp.int32, 16) == 0, 1.0, 0.0).astype(jnp.float32)

        def orow(j, _):
            pay[j, pl.ds(16, 16)] = onehot
            return 0
        lax.fori_loop(0, CH, orow, 0)

        w = c * 16 + s

        def chunk(i, _):
            off = w * ept + i * CH
            pltpu.sync_copy(dst_hbm.at[pl.ds(off, CH)], dst_v)
            pltpu.sync_copy(ea_hbm.at[pl.ds(off, CH)], ea_v)

            def ebody(j, _):
                pay[j, pl.ds(0, 16)] = ea_v[j, pl.ds(0, 16)]
                return 0
            lax.fori_loop(0, CH, ebody, 0)
            pltpu.sync_copy(pay, acc_sh.at[dst_v], add=True)
            return 0
        lax.fori_loop(0, ept // CH, chunk, 0)
        plsc.subcore_barrier()
        for r in range(RPT // CH):
            sl = pl.ds(s * RPT + r * CH, CH)
            pltpu.sync_copy(acc_sh.at[sl], out_hbm.at[c].at[sl])

    return k(dst_p, ea_p)


def _sc_edge(src_p, dst_p, ae, asrc_t, adst_t, xp_slab):
    ept = EPAD // 8
    art = 2 * NP // 16
    nch = ept // CH
    last = nch - 1

    @functools.partial(
        pl.kernel,
        out_type=jax.ShapeDtypeStruct((2, 2 * NP, ACCW), jnp.float32),
        mesh=_mesh(),
        compiler_params=pltpu.CompilerParams(
            needs_layout_passes=False, use_tc_tiling_on_sc=False),
        scratch_types=[
            pltpu.VMEM_SHARED((2 * NP, ACCW), jnp.float32),
            pltpu.VMEM((NP * 2,), jnp.float32),
            pltpu.VMEM((NP * 2,), jnp.float32),
            pltpu.VMEM((2, CH), jnp.int32),
            pltpu.VMEM((2, CH), jnp.int32),
            pltpu.VMEM((2, CH), jnp.int32),
            pltpu.VMEM((2, CH), jnp.int32),
            pltpu.VMEM((2, 2, CH), jnp.float32),
            pltpu.VMEM((2, 2, CH), jnp.float32),
            pltpu.VMEM((2, CH, 32), jnp.float32),
            pltpu.VMEM((CH, ACCW), jnp.float32),
            pltpu.SemaphoreType.DMA,
            pltpu.SemaphoreType.DMA,
            pltpu.SemaphoreType.DMA,
            pltpu.SemaphoreType.DMA,
        ],
    )
    def k(src_hbm, dst_hbm, ae_hbm, asrc_hbm, adst_hbm, xp_hbm, out_hbm,
          acc_sh, asrc_l, adst_l, src_v, dst_v, xoff_v, doff_v, ae_v, ex_v,
          xbuf, pay, sl0, sl1, sg0, sg1):
        c = lax.axis_index("c")
        s = lax.axis_index("s")
        hp = lax.rem(s, 2)
        octant = lax.div(s, 2)
        q = c * 2 + hp
        h0 = q * 2
        slds = (sl0, sl1)
        sgxs = (sg0, sg1)
        pltpu.sync_copy(asrc_hbm.at[h0], asrc_l.at[pl.ds(0, NP)])
        pltpu.sync_copy(asrc_hbm.at[h0 + 1], asrc_l.at[pl.ds(NP, NP)])
        pltpu.sync_copy(adst_hbm.at[h0], adst_l.at[pl.ds(0, NP)])
        pltpu.sync_copy(adst_hbm.at[h0 + 1], adst_l.at[pl.ds(NP, NP)])

        zv = jnp.zeros((16,), jnp.float32)

        def zrow(j, _):
            for t in range(ACCW // 16):
                pay[j, pl.ds(t * 16, 16)] = zv
            return 0
        lax.fori_loop(0, CH, zrow, 0)
        for r in range(art // CH):
            pltpu.sync_copy(pay, acc_sh.at[pl.ds(s * art + r * CH, CH)])
        plsc.subcore_barrier()

        iota16 = lax.iota(jnp.int32, 16)
        hots = [jnp.where(iota16 == h, 1.0, 0.0).astype(jnp.float32)
                for h in range(2)]

        def issue_loads(i, b):
            off = octant * ept + jnp.minimum(i, last) * CH
            pltpu.async_copy(src_hbm.at[pl.ds(off, CH)], src_v.at[b], slds[b])
            pltpu.async_copy(dst_hbm.at[pl.ds(off, CH)], dst_v.at[b], slds[b])
            pltpu.async_copy(ae_hbm.at[h0].at[pl.ds(off, CH)],
                             ae_v.at[b].at[0], slds[b])
            pltpu.async_copy(ae_hbm.at[h0 + 1].at[pl.ds(off, CH)],
                             ae_v.at[b].at[1], slds[b])

        def wait_loads(b):
            pltpu.make_async_copy(src_hbm.at[pl.ds(0, CH)], src_v.at[b],
                                  slds[b]).wait()
            pltpu.make_async_copy(dst_hbm.at[pl.ds(0, CH)], dst_v.at[b],
                                  slds[b]).wait()
            pltpu.make_async_copy(ae_hbm.at[0].at[pl.ds(0, CH)],
                                  ae_v.at[b].at[0], slds[b]).wait()
            pltpu.make_async_copy(ae_hbm.at[0].at[pl.ds(0, CH)],
                                  ae_v.at[b].at[1], slds[b]).wait()

        def front(i, b):
            wait_loads(b)
            for g in range(CH // 16):
                sl = pl.ds(g * 16, 16)
                sidx = src_v[b, sl]
                didx = dst_v[b, sl]
                for hh in range(2):
                    a = (plsc.load_gather(asrc_l, [sidx + hh * NP])
                         + plsc.load_gather(adst_l, [didx + hh * NP])
                         + ae_v[b, hh, sl])
                    a = jnp.maximum(a, 0.2 * a)
                    ex_v[b, hh, sl] = jnp.exp(a)
                xoff_v[b, sl] = sidx + q * NP
                doff_v[b, sl] = didx + hp * NP
            pltpu.async_copy(xp_hbm.at[xoff_v.at[b]], xbuf.at[b], sgxs[b])
            issue_loads(i + 1, 1 - b)

        def back(b):
            pltpu.make_async_copy(xp_hbm.at[pl.ds(0, CH)], xbuf.at[b],
                                  sgxs[b]).wait()

            def gbody(g, _):
                ev0 = ex_v[b, 0, pl.ds(g * 16, 16)]
                ev1 = ex_v[b, 1, pl.ds(g * 16, 16)]
                for jj in range(16):
                    row = g * 16 + jj
                    e0 = ev0[jj]
                    e1 = ev1[jj]
                    pay[row, pl.ds(0, 16)] = e0 * xbuf[b, row, pl.ds(0, 16)]
                    pay[row, pl.ds(16, 16)] = e1 * xbuf[b, row, pl.ds(16, 16)]
                    pay[row, pl.ds(32, 16)] = e0 * hots[0] + e1 * hots[1]
                return 0
            lax.fori_loop(0, CH // 16, gbody, 0)
            pltpu.sync_copy(pay, acc_sh.at[doff_v.at[b]], add=True)

        issue_loads(0, 0)
        front(0, 0)

        def pair(o, _):
            front(2 * o + 1, 1)
            back(0)
            front(2 * o + 2, 0)
            back(1)
            return 0
        lax.fori_loop(0, nch // 2, pair, 0)
        pltpu.make_async_copy(xp_hbm.at[pl.ds(0, CH)], xbuf.at[0],
                              sgxs[0]).wait()
        wait_loads(1)

        plsc.subcore_barrier()
        for r in range(art // CH):
            sl = pl.ds(s * art + r * CH, CH)
            pltpu.sync_copy(acc_sh.at[sl], out_hbm.at[c].at[sl])

    return k(src_p, dst_p, ae, asrc_t, adst_t, xp_slab)


def _tc_loop_finish(acc0):
    def body(a_ref, o_ref):
        ea = a_ref[0, :, :16] + a_ref[1, :, :16]
        cnt = a_ref[0, :, 16:17] + a_ref[1, :, 16:17]
        o_ref[...] = ea / jnp.maximum(cnt, 1.0)

    return pl.pallas_call(
        body,
        grid=(10,),
        in_specs=[pl.BlockSpec((2, NP // 10, LW), lambda i: (0, i, 0))],
        out_specs=pl.BlockSpec((NP // 10, EDIM), lambda i: (i, 0)),
        out_shape=jax.ShapeDtypeStruct((NP, EDIM), jnp.float32),
    )(acc0)


def _tc_edge_prep(ea_p, we1, me1, we2, me2, we3, me3):
    def body(ea_ref, w1_ref, m1_ref, w2_ref, m2_ref, w3_ref, m3_ref,
             o1_ref, o2_ref, o3_ref):
        ea = ea_ref[...]
        for w_ref, m_ref, o_ref in ((w1_ref, m1_ref, o1_ref),
                                    (w2_ref, m2_ref, o2_ref),
                                    (w3_ref, m3_ref, o3_ref)):
            wm = jnp.dot(w_ref[...], m_ref[...], preferred_element_type=jnp.float32)
            o_ref[...] = lax.dot_general(
                wm, ea, (((0,), (1,)), ((), ())),
                preferred_element_type=jnp.float32)

    bm = 4096
    wspec = pl.BlockSpec((EDIM, 128), lambda i: (0, 0))
    mspec = pl.BlockSpec((128, 8), lambda i: (0, 0))
    ospec = pl.BlockSpec((8, bm), lambda i: (0, i))
    outs = pl.pallas_call(
        body,
        grid=(EPAD // bm,),
        in_specs=[pl.BlockSpec((bm, EDIM), lambda i: (i, 0)),
                  wspec, mspec, wspec, mspec, wspec, mspec],
        out_specs=[ospec, ospec, ospec],
        out_shape=[jax.ShapeDtypeStruct((8, EPAD), jnp.float32)] * 3,
    )(ea_p, we1, me1, we2, me2, we3, me3)
    return outs


def _tc_prep(h, w, msrc, mdst, la, we, me):
    def body(h_ref, w_ref, ms_ref, md_ref, la_ref, we_ref, me_ref,
             xp_ref, xps_ref, as_ref, ad_ref, exl_ref):
        xp = jnp.dot(h_ref[...], w_ref[...], preferred_element_type=jnp.float32)
        asrc = jnp.dot(xp, ms_ref[...], preferred_element_type=jnp.float32)
        adst = jnp.dot(xp, md_ref[...], preferred_element_type=jnp.float32)
        wm = jnp.dot(we_ref[...], me_ref[...], preferred_element_type=jnp.float32)
        aeloop = jnp.dot(la_ref[...], wm, preferred_element_type=jnp.float32)
        al = asrc + adst + aeloop
        al = jnp.maximum(al, 0.2 * al)
        exl_ref[...] = jnp.exp(al)
        xp_ref[...] = xp
        for qq in range(4):
            xps_ref[qq] = xp[:, qq * 32:(qq + 1) * 32]
        as_ref[...] = lax.dot_general(
            ms_ref[...], xp, (((0,), (1,)), ((), ())),
            preferred_element_type=jnp.float32)
        ad_ref[...] = lax.dot_general(
            md_ref[...], xp, (((0,), (1,)), ((), ())),
            preferred_element_type=jnp.float32)

    bm = NP // 10
    return pl.pallas_call(
        body,
        grid=(10,),
        in_specs=[pl.BlockSpec((bm, 128), lambda i: (i, 0)),
                  pl.BlockSpec((128, 128), lambda i: (0, 0)),
                  pl.BlockSpec((128, 8), lambda i: (0, 0)),
                  pl.BlockSpec((128, 8), lambda i: (0, 0)),
                  pl.BlockSpec((bm, EDIM), lambda i: (i, 0)),
                  pl.BlockSpec((EDIM, 128), lambda i: (0, 0)),
                  pl.BlockSpec((128, 8), lambda i: (0, 0))],
        out_specs=[pl.BlockSpec((bm, 128), lambda i: (i, 0)),
                   pl.BlockSpec((4, bm, 32), lambda i: (0, i, 0)),
                   pl.BlockSpec((8, bm), lambda i: (0, i)),
                   pl.BlockSpec((8, bm), lambda i: (0, i)),
                   pl.BlockSpec((bm, 8), lambda i: (i, 0))],
        out_shape=[jax.ShapeDtypeStruct((NP, 128), jnp.float32),
                   jax.ShapeDtypeStruct((4, NP, 32), jnp.float32),
                   jax.ShapeDtypeStruct((8, NP), jnp.float32),
                   jax.ShapeDtypeStruct((8, NP), jnp.float32),
                   jax.ShapeDtypeStruct((NP, 8), jnp.float32)],
    )(h, w, msrc, mdst, la, we, me)


def _tc_norm(acc, exl, xp_sc, b, k4, do_elu):
    def body(acc_ref, exl_ref, xp_ref, b_ref, k4_ref, o_ref):
        k4 = k4_ref[...]
        for sc in range(2):
            exl_sc = exl_ref[:, sc * 4:(sc + 1) * 4]
            e64 = jnp.dot(exl_sc, k4, preferred_element_type=jnp.float32)
            xp = xp_ref[:, sc * 64:(sc + 1) * 64]
            num = jnp.concatenate([acc_ref[sc, 0, :, :32],
                                   acc_ref[sc, 1, :, :32]], axis=1)
            num = num + e64 * xp
            den4 = jnp.concatenate([acc_ref[sc, 0, :, 32:34],
                                    acc_ref[sc, 1, :, 32:34]], axis=1) + exl_sc
            den = jnp.dot(den4, k4, preferred_element_type=jnp.float32)
            o = num / den + b_ref[0, sc * 64:(sc + 1) * 64]
            if do_elu:
                o = jnp.where(o > 0, o, jnp.exp(jnp.minimum(o, 0.0)) - 1.0)
            o_ref[:, sc * 64:(sc + 1) * 64] = o

    bm = NP // 10
    return pl.pallas_call(
        body,
        grid=(10,),
        in_specs=[pl.BlockSpec((2, 2, bm, ACCW), lambda i: (0, 0, i, 0)),
                  pl.BlockSpec((bm, 8), lambda i: (i, 0)),
                  pl.BlockSpec((bm, 128), lambda i: (i, 0)),
                  pl.BlockSpec((1, 128), lambda i: (0, 0)),
                  pl.BlockSpec((4, 64), lambda i: (0, 0))],
        out_specs=pl.BlockSpec((bm, 128), lambda i: (i, 0)),
        out_shape=jax.ShapeDtypeStruct((NP, 128), jnp.float32),
    )(acc, exl, xp_sc, b, k4)


def _tc_pool(h3, p):
    def body(p_ref, h_ref, o_ref):
        pm = p_ref[...]
        s = lax.dot_general(pm, h_ref[...], (((0,), (0,)), ((), ())),
                            preferred_element_type=jnp.float32)
        cnt = jnp.sum(pm, axis=0)[:, None]
        o_ref[...] = s / jnp.maximum(cnt, 1.0)

    return pl.pallas_call(
        body,
        in_specs=[pl.BlockSpec((NP, G), lambda: (0, 0)),
                  pl.BlockSpec((NP, 128), lambda: (0, 0))],
        out_specs=pl.BlockSpec((G, 128), lambda: (0, 0)),
        out_shape=jax.ShapeDtypeStruct((G, 128), jnp.float32),
    )(p, h3)


def _mask8(a):
    return (jnp.eye(8, dtype=jnp.float32)[:, None, :] * a[0][:, :, None]).reshape(128, 8)


def _mask1(a):
    return jnp.tile(a[0, 0][:, None], (1, 8))


def kernel(x, edge_index, edge_attr, batch, W1, a_src1, a_dst1, We1, a_e1, b1,
           W2, a_src2, a_dst2, We2, a_e2, b2, W3, a_src3, a_dst3, We3, a_e3, b3):
    f32 = jnp.float32
    src = edge_index[0]
    dst = edge_index[1]
    epad = EPAD - E
    src_p = jnp.concatenate([src, jnp.full((epad,), N, jnp.int32)])
    dst_p = jnp.concatenate([dst, jnp.full((epad,), N, jnp.int32)])
    ea_p = jnp.concatenate([edge_attr, jnp.zeros((epad, EDIM), f32)])
    x_p = jnp.concatenate([x, jnp.zeros((NP - N, DIN), f32)])

    p = (batch[:, None] == jnp.arange(G, dtype=jnp.int32)[None, :]).astype(f32)
    p = jnp.concatenate([p, jnp.zeros((NP - N, G), f32)])

    k4 = (jnp.eye(4, dtype=f32)[:, :, None] * jnp.ones((1, 1, 16), f32)).reshape(4, 64)

    msrc = (_mask8(a_src1), _mask8(a_src2), _mask1(a_src3))
    mdst = (_mask8(a_dst1), _mask8(a_dst2), _mask1(a_dst3))
    me = (_mask8(a_e1), _mask8(a_e2), _mask1(a_e3))
    ws = (W1, W2, W3)
    wes = (We1, We2, We3)
    bs = (b1.reshape(1, 128), b2.reshape(1, 128), b3.reshape(1, 128))

    acc0 = _sc_loop_attr(dst_p, ea_p)
    la = _tc_loop_finish(acc0)
    aes = _tc_edge_prep(ea_p, We1, me[0], We2, me[1], We3, me[2])

    h = x_p
    for l in range(3):
        xp, xps, asrc_t, adst_t, exl = _tc_prep(h, ws[l], msrc[l], mdst[l], la,
                                                wes[l], me[l])
        acc = _sc_edge(src_p, dst_p, aes[l], asrc_t, adst_t,
                       xps.reshape(4 * NP, 32))
        h = _tc_norm(acc.reshape(2, 2, NP, ACCW), exl, xp, bs[l], k4,
                     do_elu=(l < 2))

    out = _tc_pool(h, p)
    return out


# speedup vs baseline: 40.7688x; 1.0506x over previous
import functools

import jax
import jax.numpy as jnp
from jax import lax
from jax.experimental import pallas as pl
from jax.experimental.pallas import tpu as pltpu
from jax.experimental.pallas import tpu_sc as plsc

N = 10000; E = 320000; DIN = 128; DOUT = 128; HID = 16; H1 = 8; H3 = 1; EDIM = 16; G = 64

NP = 10240
CH = 128
EPAD = 323584
ACCW = 48
LW = 32
RPT = NP // 16

_mesh = functools.partial(
    plsc.VectorSubcoreMesh,
    core_axis_name="c", subcore_axis_name="s", num_cores=2, num_subcores=16)


def _sc_loop_attr(dst_p, ea_p):
    ept = EPAD // 32

    @functools.partial(
        pl.kernel,
        out_type=jax.ShapeDtypeStruct((2, NP, LW), jnp.float32),
        mesh=_mesh(),
        compiler_params=pltpu.CompilerParams(needs_layout_passes=False),
        scratch_types=[
            pltpu.VMEM_SHARED((NP, LW), jnp.float32),
            pltpu.VMEM((CH,), jnp.int32),
            pltpu.VMEM((CH, EDIM), jnp.float32),
            pltpu.VMEM((CH, LW), jnp.float32),
        ],
    )
    def k(dst_hbm, ea_hbm, out_hbm, acc_sh, dst_v, ea_v, pay):
        c = lax.axis_index("c")
        s = lax.axis_index("s")

        zv = jnp.zeros((16,), jnp.float32)

        def zrow(j, _):
            for t in range(LW // 16):
                pay[j, pl.ds(t * 16, 16)] = zv
            return 0
        lax.fori_loop(0, CH, zrow, 0)
        for r in range(RPT // CH):
            pltpu.sync_copy(pay, acc_sh.at[pl.ds(s * RPT + r * CH, CH)])
        plsc.subcore_barrier()

        onehot = jnp.where(lax.iota(jnp.int32, 16) == 0, 1.0, 0.0).astype(jnp.float32)

        def orow(j, _):
            pay[j, pl.ds(16, 16)] = onehot
            return 0
        lax.fori_loop(0, CH, orow, 0)

        w = c * 16 + s

        def chunk(i, _):
            off = w * ept + i * CH
            pltpu.sync_copy(dst_hbm.at[pl.ds(off, CH)], dst_v)
            pltpu.sync_copy(ea_hbm.at[pl.ds(off, CH)], ea_v)

            def ebody(j, _):
                pay[j, pl.ds(0, 16)] = ea_v[j, pl.ds(0, 16)]
                return 0
            lax.fori_loop(0, CH, ebody, 0)
            pltpu.sync_copy(pay, acc_sh.at[dst_v], add=True)
            return 0
        lax.fori_loop(0, ept // CH, chunk, 0)
        plsc.subcore_barrier()
        for r in range(RPT // CH):
            sl = pl.ds(s * RPT + r * CH, CH)
            pltpu.sync_copy(acc_sh.at[sl], out_hbm.at[c].at[sl])

    return k(dst_p, ea_p)


def _sc_edge(src_p, dst_p, ae, asrc_t, adst_t, xp_slab):
    ept = EPAD // 8
    art = 2 * NP // 16
    nch = ept // CH
    last = nch - 1

    @functools.partial(
        pl.kernel,
        out_type=jax.ShapeDtypeStruct((2, 2 * NP, ACCW), jnp.float32),
        mesh=_mesh(),
        compiler_params=pltpu.CompilerParams(
            needs_layout_passes=False, use_tc_tiling_on_sc=False),
        scratch_types=[
            pltpu.VMEM_SHARED((2 * NP, ACCW), jnp.float32),
            pltpu.VMEM((NP * 2,), jnp.float32),
            pltpu.VMEM((NP * 2,), jnp.float32),
            pltpu.VMEM((2, CH), jnp.int32),
            pltpu.VMEM((2, CH), jnp.int32),
            pltpu.VMEM((2, CH), jnp.int32),
            pltpu.VMEM((2, CH), jnp.int32),
            pltpu.VMEM((2, CH), jnp.int32),
            pltpu.VMEM((2, 2, CH), jnp.float32),
            pltpu.VMEM((2, 2, CH), jnp.float32),
            pltpu.VMEM((2, CH, 32), jnp.float32),
            pltpu.VMEM((2, CH, ACCW), jnp.float32),
            pltpu.SemaphoreType.DMA,
            pltpu.SemaphoreType.DMA,
            pltpu.SemaphoreType.DMA,
            pltpu.SemaphoreType.DMA,
            pltpu.SemaphoreType.DMA,
            pltpu.SemaphoreType.DMA,
        ],
    )
    def k(src_hbm, dst_hbm, ae_hbm, asrc_hbm, adst_hbm, xp_hbm, out_hbm,
          acc_sh, asrc_l, adst_l, src_v, dst_v, xoff_v, doff_v, sdoff_v, ae_v,
          ex_v, xbuf, pay, sl0, sl1, sg0, sg1, ss0, ss1):
        c = lax.axis_index("c")
        s = lax.axis_index("s")
        hp = lax.rem(s, 2)
        octant = lax.div(s, 2)
        q = c * 2 + hp
        h0 = q * 2
        slds = (sl0, sl1)
        sgxs = (sg0, sg1)
        pltpu.sync_copy(asrc_hbm.at[h0], asrc_l.at[pl.ds(0, NP)])
        pltpu.sync_copy(asrc_hbm.at[h0 + 1], asrc_l.at[pl.ds(NP, NP)])
        pltpu.sync_copy(adst_hbm.at[h0], adst_l.at[pl.ds(0, NP)])
        pltpu.sync_copy(adst_hbm.at[h0 + 1], adst_l.at[pl.ds(NP, NP)])

        zv = jnp.zeros((16,), jnp.float32)

        def zrow(j, _):
            for t in range(ACCW // 16):
                pay[0, j, pl.ds(t * 16, 16)] = zv
            return 0
        lax.fori_loop(0, CH, zrow, 0)
        for r in range(art // CH):
            pltpu.sync_copy(pay.at[0], acc_sh.at[pl.ds(s * art + r * CH, CH)])
        plsc.subcore_barrier()

        iota16 = lax.iota(jnp.int32, 16)
        hots = [jnp.where(iota16 == h, 1.0, 0.0).astype(jnp.float32)
                for h in range(2)]

        def issue_loads(i, b):
            off = octant * ept + jnp.minimum(i, last) * CH
            pltpu.async_copy(src_hbm.at[pl.ds(off, CH)], src_v.at[b], slds[b])
            pltpu.async_copy(dst_hbm.at[pl.ds(off, CH)], dst_v.at[b], slds[b])
            pltpu.async_copy(ae_hbm.at[h0].at[pl.ds(off, CH)],
                             ae_v.at[b].at[0], slds[b])
            pltpu.async_copy(ae_hbm.at[h0 + 1].at[pl.ds(off, CH)],
                             ae_v.at[b].at[1], slds[b])

        def wait_loads(b):
            pltpu.make_async_copy(src_hbm.at[pl.ds(0, CH)], src_v.at[b],
                                  slds[b]).wait()
            pltpu.make_async_copy(dst_hbm.at[pl.ds(0, CH)], dst_v.at[b],
                                  slds[b]).wait()
            pltpu.make_async_copy(ae_hbm.at[0].at[pl.ds(0, CH)],
                                  ae_v.at[b].at[0], slds[b]).wait()
            pltpu.make_async_copy(ae_hbm.at[0].at[pl.ds(0, CH)],
                                  ae_v.at[b].at[1], slds[b]).wait()

        def front(i, b):
            wait_loads(b)
            for g in range(CH // 16):
                sl = pl.ds(g * 16, 16)
                sidx = src_v[b, sl]
                didx = dst_v[b, sl]
                for hh in range(2):
                    a = (plsc.load_gather(asrc_l, [sidx + hh * NP])
                         + plsc.load_gather(adst_l, [didx + hh * NP])
                         + ae_v[b, hh, sl])
                    a = jnp.maximum(a, 0.2 * a)
                    ex_v[b, hh, sl] = jnp.exp(a)
                xoff_v[b, sl] = sidx + q * NP
                doff_v[b, sl] = didx + hp * NP
            pltpu.async_copy(xp_hbm.at[xoff_v.at[b]], xbuf.at[b], sgxs[b])
            issue_loads(i + 1, 1 - b)

        def back_issue(b):
            pltpu.make_async_copy(xp_hbm.at[pl.ds(0, CH)], xbuf.at[b],
                                  sgxs[b]).wait()

            def gbody(g, _):
                sl = pl.ds(g * 16, 16)
                sdoff_v[b, sl] = doff_v[b, sl]
                ev0 = ex_v[b, 0, sl]
                ev1 = ex_v[b, 1, sl]
                for jj in range(16):
                    row = g * 16 + jj
                    e0 = ev0[jj]
                    e1 = ev1[jj]
                    pay[b, row, pl.ds(0, 16)] = e0 * xbuf[b, row, pl.ds(0, 16)]
                    pay[b, row, pl.ds(16, 16)] = e1 * xbuf[b, row, pl.ds(16, 16)]
                    pay[b, row, pl.ds(32, 16)] = e0 * hots[0] + e1 * hots[1]
                return 0
            lax.fori_loop(0, CH // 16, gbody, 0)
            return pltpu.async_copy(pay.at[b], acc_sh.at[sdoff_v.at[b]],
                                    (ss0, ss1)[b], add=True)

        issue_loads(0, 0)
        front(0, 0)

        def pair(o, _):
            front(2 * o + 1, 1)
            h0 = back_issue(0)
            front(2 * o + 2, 0)
            h1 = back_issue(1)
            h0.wait()
            h1.wait()
            return 0
        lax.fori_loop(0, nch // 2, pair, 0)
        pltpu.make_async_copy(xp_hbm.at[pl.ds(0, CH)], xbuf.at[0],
                              sgxs[0]).wait()
        wait_loads(1)

        plsc.subcore_barrier()
        for r in range(art // CH):
            sl = pl.ds(s * art + r * CH, CH)
            pltpu.sync_copy(acc_sh.at[sl], out_hbm.at[c].at[sl])

    return k(src_p, dst_p, ae, asrc_t, adst_t, xp_slab)


def _tc_loop_finish(acc0):
    def body(a_ref, o_ref):
        ea = a_ref[0, :, :16] + a_ref[1, :, :16]
        cnt = a_ref[0, :, 16:17] + a_ref[1, :, 16:17]
        o_ref[...] = ea / jnp.maximum(cnt, 1.0)

    return pl.pallas_call(
        body,
        grid=(10,),
        in_specs=[pl.BlockSpec((2, NP // 10, LW), lambda i: (0, i, 0))],
        out_specs=pl.BlockSpec((NP // 10, EDIM), lambda i: (i, 0)),
        out_shape=jax.ShapeDtypeStruct((NP, EDIM), jnp.float32),
    )(acc0)


def _tc_edge_prep(ea_p, we1, me1, we2, me2, we3, me3):
    def body(ea_ref, w1_ref, m1_ref, w2_ref, m2_ref, w3_ref, m3_ref,
             o1_ref, o2_ref, o3_ref):
        ea = ea_ref[...]
        for w_ref, m_ref, o_ref in ((w1_ref, m1_ref, o1_ref),
                                    (w2_ref, m2_ref, o2_ref),
                                    (w3_ref, m3_ref, o3_ref)):
            wm = jnp.dot(w_ref[...], m_ref[...], preferred_element_type=jnp.float32)
            o_ref[...] = lax.dot_general(
                wm, ea, (((0,), (1,)), ((), ())),
                preferred_element_type=jnp.float32)

    bm = 4096
    wspec = pl.BlockSpec((EDIM, 128), lambda i: (0, 0))
    mspec = pl.BlockSpec((128, 8), lambda i: (0, 0))
    ospec = pl.BlockSpec((8, bm), lambda i: (0, i))
    outs = pl.pallas_call(
        body,
        grid=(EPAD // bm,),
        in_specs=[pl.BlockSpec((bm, EDIM), lambda i: (i, 0)),
                  wspec, mspec, wspec, mspec, wspec, mspec],
        out_specs=[ospec, ospec, ospec],
        out_shape=[jax.ShapeDtypeStruct((8, EPAD), jnp.float32)] * 3,
    )(ea_p, we1, me1, we2, me2, we3, me3)
    return outs


def _tc_prep(h, w, msrc, mdst, la, we, me):
    def body(h_ref, w_ref, ms_ref, md_ref, la_ref, we_ref, me_ref,
             xp_ref, xps_ref, as_ref, ad_ref, exl_ref):
        xp = jnp.dot(h_ref[...], w_ref[...], preferred_element_type=jnp.float32)
        asrc = jnp.dot(xp, ms_ref[...], preferred_element_type=jnp.float32)
        adst = jnp.dot(xp, md_ref[...], preferred_element_type=jnp.float32)
        wm = jnp.dot(we_ref[...], me_ref[...], preferred_element_type=jnp.float32)
        aeloop = jnp.dot(la_ref[...], wm, preferred_element_type=jnp.float32)
        al = asrc + adst + aeloop
        al = jnp.maximum(al, 0.2 * al)
        exl_ref[...] = jnp.exp(al)
        xp_ref[...] = xp
        for qq in range(4):
            xps_ref[qq] = xp[:, qq * 32:(qq + 1) * 32]
        as_ref[...] = lax.dot_general(
            ms_ref[...], xp, (((0,), (1,)), ((), ())),
            preferred_element_type=jnp.float32)
        ad_ref[...] = lax.dot_general(
            md_ref[...], xp, (((0,), (1,)), ((), ())),
            preferred_element_type=jnp.float32)

    bm = NP // 10
    return pl.pallas_call(
        body,
        grid=(10,),
        in_specs=[pl.BlockSpec((bm, 128), lambda i: (i, 0)),
                  pl.BlockSpec((128, 128), lambda i: (0, 0)),
                  pl.BlockSpec((128, 8), lambda i: (0, 0)),
                  pl.BlockSpec((128, 8), lambda i: (0, 0)),
                  pl.BlockSpec((bm, EDIM), lambda i: (i, 0)),
                  pl.BlockSpec((EDIM, 128), lambda i: (0, 0)),
                  pl.BlockSpec((128, 8), lambda i: (0, 0))],
        out_specs=[pl.BlockSpec((bm, 128), lambda i: (i, 0)),
                   pl.BlockSpec((4, bm, 32), lambda i: (0, i, 0)),
                   pl.BlockSpec((8, bm), lambda i: (0, i)),
                   pl.BlockSpec((8, bm), lambda i: (0, i)),
                   pl.BlockSpec((bm, 8), lambda i: (i, 0))],
        out_shape=[jax.ShapeDtypeStruct((NP, 128), jnp.float32),
                   jax.ShapeDtypeStruct((4, NP, 32), jnp.float32),
                   jax.ShapeDtypeStruct((8, NP), jnp.float32),
                   jax.ShapeDtypeStruct((8, NP), jnp.float32),
                   jax.ShapeDtypeStruct((NP, 8), jnp.float32)],
    )(h, w, msrc, mdst, la, we, me)


def _tc_norm(acc, exl, xp_sc, b, k4, do_elu):
    def body(acc_ref, exl_ref, xp_ref, b_ref, k4_ref, o_ref):
        k4 = k4_ref[...]
        for sc in range(2):
            exl_sc = exl_ref[:, sc * 4:(sc + 1) * 4]
            e64 = jnp.dot(exl_sc, k4, preferred_element_type=jnp.float32)
            xp = xp_ref[:, sc * 64:(sc + 1) * 64]
            num = jnp.concatenate([acc_ref[sc, 0, :, :32],
                                   acc_ref[sc, 1, :, :32]], axis=1)
            num = num + e64 * xp
            den4 = jnp.concatenate([acc_ref[sc, 0, :, 32:34],
                                    acc_ref[sc, 1, :, 32:34]], axis=1) + exl_sc
            den = jnp.dot(den4, k4, preferred_element_type=jnp.float32)
            o = num / den + b_ref[0, sc * 64:(sc + 1) * 64]
            if do_elu:
                o = jnp.where(o > 0, o, jnp.exp(jnp.minimum(o, 0.0)) - 1.0)
            o_ref[:, sc * 64:(sc + 1) * 64] = o

    bm = NP // 10
    return pl.pallas_call(
        body,
        grid=(10,),
        in_specs=[pl.BlockSpec((2, 2, bm, ACCW), lambda i: (0, 0, i, 0)),
                  pl.BlockSpec((bm, 8), lambda i: (i, 0)),
                  pl.BlockSpec((bm, 128), lambda i: (i, 0)),
                  pl.BlockSpec((1, 128), lambda i: (0, 0)),
                  pl.BlockSpec((4, 64), lambda i: (0, 0))],
        out_specs=pl.BlockSpec((bm, 128), lambda i: (i, 0)),
        out_shape=jax.ShapeDtypeStruct((NP, 128), jnp.float32),
    )(acc, exl, xp_sc, b, k4)


def _tc_pool(h3, p):
    def body(p_ref, h_ref, o_ref):
        pm = p_ref[...]
        s = lax.dot_general(pm, h_ref[...], (((0,), (0,)), ((), ())),
                            preferred_element_type=jnp.float32)
        cnt = jnp.sum(pm, axis=0)[:, None]
        o_ref[...] = s / jnp.maximum(cnt, 1.0)

    return pl.pallas_call(
        body,
        in_specs=[pl.BlockSpec((NP, G), lambda: (0, 0)),
                  pl.BlockSpec((NP, 128), lambda: (0, 0))],
        out_specs=pl.BlockSpec((G, 128), lambda: (0, 0)),
        out_shape=jax.ShapeDtypeStruct((G, 128), jnp.float32),
    )(p, h3)


def _mask8(a):
    return (jnp.eye(8, dtype=jnp.float32)[:, None, :] * a[0][:, :, None]).reshape(128, 8)


def _mask1(a):
    return jnp.tile(a[0, 0][:, None], (1, 8))


def kernel(x, edge_index, edge_attr, batch, W1, a_src1, a_dst1, We1, a_e1, b1,
           W2, a_src2, a_dst2, We2, a_e2, b2, W3, a_src3, a_dst3, We3, a_e3, b3):
    f32 = jnp.float32
    src = edge_index[0]
    dst = edge_index[1]
    epad = EPAD - E
    src_p = jnp.concatenate([src, jnp.full((epad,), N, jnp.int32)])
    dst_p = jnp.concatenate([dst, jnp.full((epad,), N, jnp.int32)])
    ea_p = jnp.concatenate([edge_attr, jnp.zeros((epad, EDIM), f32)])
    x_p = jnp.concatenate([x, jnp.zeros((NP - N, DIN), f32)])

    p = (batch[:, None] == jnp.arange(G, dtype=jnp.int32)[None, :]).astype(f32)
    p = jnp.concatenate([p, jnp.zeros((NP - N, G), f32)])

    k4 = (jnp.eye(4, dtype=f32)[:, :, None] * jnp.ones((1, 1, 16), f32)).reshape(4, 64)

    msrc = (_mask8(a_src1), _mask8(a_src2), _mask1(a_src3))
    mdst = (_mask8(a_dst1), _mask8(a_dst2), _mask1(a_dst3))
    me = (_mask8(a_e1), _mask8(a_e2), _mask1(a_e3))
    ws = (W1, W2, W3)
    wes = (We1, We2, We3)
    bs = (b1.reshape(1, 128), b2.reshape(1, 128), b3.reshape(1, 128))

    acc0 = _sc_loop_attr(dst_p, ea_p)
    la = _tc_loop_finish(acc0)
    aes = _tc_edge_prep(ea_p, We1, me[0], We2, me[1], We3, me[2])

    h = x_p
    for l in range(3):
        xp, xps, asrc_t, adst_t, exl = _tc_prep(h, ws[l], msrc[l], mdst[l], la,
                                                wes[l], me[l])
        acc = _sc_edge(src_p, dst_p, aes[l], asrc_t, adst_t,
                       xps.reshape(4 * NP, 32))
        h = _tc_norm(acc.reshape(2, 2, NP, ACCW), exl, xp, bs[l], k4,
                     do_elu=(l < 2))

    out = _tc_pool(h, p)
    return out


# speedup vs baseline: 41.9377x; 1.0287x over previous
import functools

import jax
import jax.numpy as jnp
from jax import lax
from jax.experimental import pallas as pl
from jax.experimental.pallas import tpu as pltpu
from jax.experimental.pallas import tpu_sc as plsc

N = 10000; E = 320000; DIN = 128; DOUT = 128; HID = 16; H1 = 8; H3 = 1; EDIM = 16; G = 64

NP = 10240
CH = 128
EPAD = 323584
ACCW = 48
LW = 32
RPT = NP // 16

_mesh = functools.partial(
    plsc.VectorSubcoreMesh,
    core_axis_name="c", subcore_axis_name="s", num_cores=2, num_subcores=16)


def _sc_loop_attr(dst_p, ea_p):
    ept = EPAD // 32
    ch0 = 128
    nch0 = ept // ch0

    @functools.partial(
        pl.kernel,
        out_type=[jax.ShapeDtypeStruct((2, NP, EDIM), jnp.float32),
                  jax.ShapeDtypeStruct((2, NP, 16), jnp.float32)],
        mesh=_mesh(),
        compiler_params=pltpu.CompilerParams(needs_layout_passes=False),
        scratch_types=[
            pltpu.VMEM_SHARED((NP, EDIM), jnp.float32),
            pltpu.VMEM_SHARED((NP, 16), jnp.float32),
            pltpu.VMEM((2, ch0), jnp.int32),
            pltpu.VMEM((2, ch0, EDIM), jnp.float32),
            pltpu.VMEM((ch0, 16), jnp.float32),
            pltpu.SemaphoreType.DMA,
            pltpu.SemaphoreType.DMA,
        ],
    )
    def k(dst_hbm, ea_hbm, outa_hbm, outb_hbm, acca_sh, accb_sh, dst_v, pay,
          ones, sl0, sl1):
        c = lax.axis_index("c")
        s = lax.axis_index("s")
        slds = (sl0, sl1)

        zv = jnp.zeros((16,), jnp.float32)
        onehot = jnp.where(lax.iota(jnp.int32, 16) == 0, 1.0, 0.0).astype(jnp.float32)

        def zrow(j, _):
            ones[j, pl.ds(0, 16)] = zv
            return 0
        lax.fori_loop(0, ch0, zrow, 0)
        rpt = NP // 16
        for r in range((rpt + ch0 - 1) // ch0):
            base = jnp.minimum(s * rpt + r * ch0, NP - ch0)
            pltpu.sync_copy(ones, accb_sh.at[pl.ds(base, ch0)])
            pltpu.sync_copy(ones, acca_sh.at[pl.ds(base, ch0)])

        def orow(j, _):
            ones[j, pl.ds(0, 16)] = onehot
            return 0
        lax.fori_loop(0, ch0, orow, 0)
        plsc.subcore_barrier()

        w = c * 16 + s

        def issue_loads(i, b):
            off = w * ept + jnp.minimum(i, nch0 - 1) * ch0
            pltpu.async_copy(dst_hbm.at[pl.ds(off, ch0)], dst_v.at[b], slds[b])
            pltpu.async_copy(ea_hbm.at[pl.ds(off, ch0)], pay.at[b], slds[b])

        def step(b):
            pltpu.make_async_copy(dst_hbm.at[pl.ds(0, ch0)], dst_v.at[b],
                                  slds[b]).wait()
            pltpu.make_async_copy(ea_hbm.at[pl.ds(0, ch0)], pay.at[b],
                                  slds[b]).wait()

        issue_loads(0, 0)

        def pairc(o, _):
            step(0)
            issue_loads(2 * o + 1, 1)
            pltpu.sync_copy(pay.at[0], acca_sh.at[dst_v.at[0]], add=True)
            pltpu.sync_copy(ones, accb_sh.at[dst_v.at[0]], add=True)
            step(1)
            issue_loads(2 * o + 2, 0)
            pltpu.sync_copy(pay.at[1], acca_sh.at[dst_v.at[1]], add=True)
            pltpu.sync_copy(ones, accb_sh.at[dst_v.at[1]], add=True)
            return 0
        lax.fori_loop(0, nch0 // 2, pairc, 0)
        step(0)
        pltpu.sync_copy(pay.at[0], acca_sh.at[dst_v.at[0]], add=True)
        pltpu.sync_copy(ones, accb_sh.at[dst_v.at[0]], add=True)

        plsc.subcore_barrier()
        for r in range((rpt + ch0 - 1) // ch0):
            base = jnp.minimum(s * rpt + r * ch0, NP - ch0)
            sl = pl.ds(base, ch0)
            pltpu.sync_copy(acca_sh.at[sl], outa_hbm.at[c].at[sl])
            pltpu.sync_copy(accb_sh.at[sl], outb_hbm.at[c].at[sl])

    return k(dst_p, ea_p)


def _sc_edge(src_p, dst_p, ae, asrc_t, adst_t, xp_slab):
    ept = EPAD // 8
    art = 2 * NP // 16
    nch = ept // CH
    last = nch - 1

    @functools.partial(
        pl.kernel,
        out_type=jax.ShapeDtypeStruct((2, 2 * NP, ACCW), jnp.float32),
        mesh=_mesh(),
        compiler_params=pltpu.CompilerParams(
            needs_layout_passes=False, use_tc_tiling_on_sc=False),
        scratch_types=[
            pltpu.VMEM_SHARED((2 * NP, ACCW), jnp.float32),
            pltpu.VMEM((NP * 2,), jnp.float32),
            pltpu.VMEM((NP * 2,), jnp.float32),
            pltpu.VMEM((2, CH), jnp.int32),
            pltpu.VMEM((2, CH), jnp.int32),
            pltpu.VMEM((2, CH), jnp.int32),
            pltpu.VMEM((2, CH), jnp.int32),
            pltpu.VMEM((2, CH), jnp.int32),
            pltpu.VMEM((2, 2, CH), jnp.float32),
            pltpu.VMEM((2, 2, CH), jnp.float32),
            pltpu.VMEM((2, CH, 32), jnp.float32),
            pltpu.VMEM((2, CH, ACCW), jnp.float32),
            pltpu.SemaphoreType.DMA,
            pltpu.SemaphoreType.DMA,
            pltpu.SemaphoreType.DMA,
            pltpu.SemaphoreType.DMA,
            pltpu.SemaphoreType.DMA,
            pltpu.SemaphoreType.DMA,
        ],
    )
    def k(src_hbm, dst_hbm, ae_hbm, asrc_hbm, adst_hbm, xp_hbm, out_hbm,
          acc_sh, asrc_l, adst_l, src_v, dst_v, xoff_v, doff_v, sdoff_v, ae_v,
          ex_v, xbuf, pay, sl0, sl1, sg0, sg1, ss0, ss1):
        c = lax.axis_index("c")
        s = lax.axis_index("s")
        hp = lax.rem(s, 2)
        octant = lax.div(s, 2)
        q = c * 2 + hp
        h0 = q * 2
        slds = (sl0, sl1)
        sgxs = (sg0, sg1)
        pltpu.sync_copy(asrc_hbm.at[h0], asrc_l.at[pl.ds(0, NP)])
        pltpu.sync_copy(asrc_hbm.at[h0 + 1], asrc_l.at[pl.ds(NP, NP)])
        pltpu.sync_copy(adst_hbm.at[h0], adst_l.at[pl.ds(0, NP)])
        pltpu.sync_copy(adst_hbm.at[h0 + 1], adst_l.at[pl.ds(NP, NP)])

        zv = jnp.zeros((16,), jnp.float32)

        def zrow(j, _):
            for t in range(ACCW // 16):
                pay[0, j, pl.ds(t * 16, 16)] = zv
            return 0
        lax.fori_loop(0, CH, zrow, 0)
        for r in range(art // CH):
            pltpu.sync_copy(pay.at[0], acc_sh.at[pl.ds(s * art + r * CH, CH)])
        plsc.subcore_barrier()

        iota16 = lax.iota(jnp.int32, 16)
        hots = [jnp.where(iota16 == h, 1.0, 0.0).astype(jnp.float32)
                for h in range(2)]

        def issue_loads(i, b):
            off = octant * ept + jnp.minimum(i, last) * CH
            pltpu.async_copy(src_hbm.at[pl.ds(off, CH)], src_v.at[b], slds[b])
            pltpu.async_copy(dst_hbm.at[pl.ds(off, CH)], dst_v.at[b], slds[b])
            pltpu.async_copy(ae_hbm.at[h0].at[pl.ds(off, CH)],
                             ae_v.at[b].at[0], slds[b])
            pltpu.async_copy(ae_hbm.at[h0 + 1].at[pl.ds(off, CH)],
                             ae_v.at[b].at[1], slds[b])

        def wait_loads(b):
            pltpu.make_async_copy(src_hbm.at[pl.ds(0, CH)], src_v.at[b],
                                  slds[b]).wait()
            pltpu.make_async_copy(dst_hbm.at[pl.ds(0, CH)], dst_v.at[b],
                                  slds[b]).wait()
            pltpu.make_async_copy(ae_hbm.at[0].at[pl.ds(0, CH)],
                                  ae_v.at[b].at[0], slds[b]).wait()
            pltpu.make_async_copy(ae_hbm.at[0].at[pl.ds(0, CH)],
                                  ae_v.at[b].at[1], slds[b]).wait()

        def front(i, b):
            wait_loads(b)
            for g in range(CH // 16):
                sl = pl.ds(g * 16, 16)
                sidx = src_v[b, sl]
                didx = dst_v[b, sl]
                for hh in range(2):
                    a = (plsc.load_gather(asrc_l, [sidx + hh * NP])
                         + plsc.load_gather(adst_l, [didx + hh * NP])
                         + ae_v[b, hh, sl])
                    a = jnp.maximum(a, 0.2 * a)
                    ex_v[b, hh, sl] = jnp.exp(a)
                xoff_v[b, sl] = sidx + q * NP
                doff_v[b, sl] = didx + hp * NP
            pltpu.async_copy(xp_hbm.at[xoff_v.at[b]], xbuf.at[b], sgxs[b])
            issue_loads(i + 1, 1 - b)

        def back_issue(b):
            pltpu.make_async_copy(xp_hbm.at[pl.ds(0, CH)], xbuf.at[b],
                                  sgxs[b]).wait()

            def gbody(g, _):
                sl = pl.ds(g * 16, 16)
                sdoff_v[b, sl] = doff_v[b, sl]
                ev0 = ex_v[b, 0, sl]
                ev1 = ex_v[b, 1, sl]
                for jj in range(16):
                    row = g * 16 + jj
                    e0 = ev0[jj]
                    e1 = ev1[jj]
                    pay[b, row, pl.ds(0, 16)] = e0 * xbuf[b, row, pl.ds(0, 16)]
                    pay[b, row, pl.ds(16, 16)] = e1 * xbuf[b, row, pl.ds(16, 16)]
                    pay[b, row, pl.ds(32, 16)] = e0 * hots[0] + e1 * hots[1]
                return 0
            lax.fori_loop(0, CH // 16, gbody, 0)
            return pltpu.async_copy(pay.at[b], acc_sh.at[sdoff_v.at[b]],
                                    (ss0, ss1)[b], add=True)

        issue_loads(0, 0)
        front(0, 0)

        def pair(o, _):
            front(2 * o + 1, 1)
            h0 = back_issue(0)
            front(2 * o + 2, 0)
            h1 = back_issue(1)
            h0.wait()
            h1.wait()
            return 0
        lax.fori_loop(0, nch // 2, pair, 0)
        pltpu.make_async_copy(xp_hbm.at[pl.ds(0, CH)], xbuf.at[0],
                              sgxs[0]).wait()
        wait_loads(1)

        plsc.subcore_barrier()
        for r in range(art // CH):
            sl = pl.ds(s * art + r * CH, CH)
            pltpu.sync_copy(acc_sh.at[sl], out_hbm.at[c].at[sl])

    return k(src_p, dst_p, ae, asrc_t, adst_t, xp_slab)


def _tc_loop_finish(acca, accb):
    def body(a_ref, b_ref, o_ref):
        ea = a_ref[0] + a_ref[1]
        cnt = b_ref[0, :, 0:1] + b_ref[1, :, 0:1]
        o_ref[...] = ea / jnp.maximum(cnt, 1.0)

    bm = NP // 10
    return pl.pallas_call(
        body,
        grid=(10,),
        in_specs=[pl.BlockSpec((2, bm, EDIM), lambda i: (0, i, 0)),
                  pl.BlockSpec((2, bm, 16), lambda i: (0, i, 0))],
        out_specs=pl.BlockSpec((bm, EDIM), lambda i: (i, 0)),
        out_shape=jax.ShapeDtypeStruct((NP, EDIM), jnp.float32),
    )(acca, accb)


def _tc_edge_prep(ea_p, we1, me1, we2, me2, we3, me3):
    def body(ea_ref, w1_ref, m1_ref, w2_ref, m2_ref, w3_ref, m3_ref,
             o1_ref, o2_ref, o3_ref):
        ea = ea_ref[...]
        for w_ref, m_ref, o_ref in ((w1_ref, m1_ref, o1_ref),
                                    (w2_ref, m2_ref, o2_ref),
                                    (w3_ref, m3_ref, o3_ref)):
            wm = jnp.dot(w_ref[...], m_ref[...], preferred_element_type=jnp.float32)
            o_ref[...] = lax.dot_general(
                wm, ea, (((0,), (1,)), ((), ())),
                preferred_element_type=jnp.float32)

    bm = 4096
    wspec = pl.BlockSpec((EDIM, 128), lambda i: (0, 0))
    mspec = pl.BlockSpec((128, 8), lambda i: (0, 0))
    ospec = pl.BlockSpec((8, bm), lambda i: (0, i))
    outs = pl.pallas_call(
        body,
        grid=(EPAD // bm,),
        in_specs=[pl.BlockSpec((bm, EDIM), lambda i: (i, 0)),
                  wspec, mspec, wspec, mspec, wspec, mspec],
        out_specs=[ospec, ospec, ospec],
        out_shape=[jax.ShapeDtypeStruct((8, EPAD), jnp.float32)] * 3,
    )(ea_p, we1, me1, we2, me2, we3, me3)
    return outs


def _tc_prep(h, w, msrc, mdst, la, we, me):
    def body(h_ref, w_ref, ms_ref, md_ref, la_ref, we_ref, me_ref,
             xp_ref, xps_ref, as_ref, ad_ref, exl_ref):
        xp = jnp.dot(h_ref[...], w_ref[...], preferred_element_type=jnp.float32)
        asrc = jnp.dot(xp, ms_ref[...], preferred_element_type=jnp.float32)
        adst = jnp.dot(xp, md_ref[...], preferred_element_type=jnp.float32)
        wm = jnp.dot(we_ref[...], me_ref[...], preferred_element_type=jnp.float32)
        aeloop = jnp.dot(la_ref[...], wm, preferred_element_type=jnp.float32)
        al = asrc + adst + aeloop
        al = jnp.maximum(al, 0.2 * al)
        exl_ref[...] = jnp.exp(al)
        xp_ref[...] = xp
        for qq in range(4):
            xps_ref[qq] = xp[:, qq * 32:(qq + 1) * 32]
        as_ref[...] = lax.dot_general(
            ms_ref[...], xp, (((0,), (1,)), ((), ())),
            preferred_element_type=jnp.float32)
        ad_ref[...] = lax.dot_general(
            md_ref[...], xp, (((0,), (1,)), ((), ())),
            preferred_element_type=jnp.float32)

    bm = NP // 10
    return pl.pallas_call(
        body,
        grid=(10,),
        in_specs=[pl.BlockSpec((bm, 128), lambda i: (i, 0)),
                  pl.BlockSpec((128, 128), lambda i: (0, 0)),
                  pl.BlockSpec((128, 8), lambda i: (0, 0)),
                  pl.BlockSpec((128, 8), lambda i: (0, 0)),
                  pl.BlockSpec((bm, EDIM), lambda i: (i, 0)),
                  pl.BlockSpec((EDIM, 128), lambda i: (0, 0)),
                  pl.BlockSpec((128, 8), lambda i: (0, 0))],
        out_specs=[pl.BlockSpec((bm, 128), lambda i: (i, 0)),
                   pl.BlockSpec((4, bm, 32), lambda i: (0, i, 0)),
                   pl.BlockSpec((8, bm), lambda i: (0, i)),
                   pl.BlockSpec((8, bm), lambda i: (0, i)),
                   pl.BlockSpec((bm, 8), lambda i: (i, 0))],
        out_shape=[jax.ShapeDtypeStruct((NP, 128), jnp.float32),
                   jax.ShapeDtypeStruct((4, NP, 32), jnp.float32),
                   jax.ShapeDtypeStruct((8, NP), jnp.float32),
                   jax.ShapeDtypeStruct((8, NP), jnp.float32),
                   jax.ShapeDtypeStruct((NP, 8), jnp.float32)],
    )(h, w, msrc, mdst, la, we, me)


def _tc_norm(acc, exl, xp_sc, b, k4, do_elu):
    def body(acc_ref, exl_ref, xp_ref, b_ref, k4_ref, o_ref):
        k4 = k4_ref[...]
        for sc in range(2):
            exl_sc = exl_ref[:, sc * 4:(sc + 1) * 4]
            e64 = jnp.dot(exl_sc, k4, preferred_element_type=jnp.float32)
            xp = xp_ref[:, sc * 64:(sc + 1) * 64]
            num = jnp.concatenate([acc_ref[sc, 0, :, :32],
                                   acc_ref[sc, 1, :, :32]], axis=1)
            num = num + e64 * xp
            den4 = jnp.concatenate([acc_ref[sc, 0, :, 32:34],
                                    acc_ref[sc, 1, :, 32:34]], axis=1) + exl_sc
            den = jnp.dot(den4, k4, preferred_element_type=jnp.float32)
            o = num / den + b_ref[0, sc * 64:(sc + 1) * 64]
            if do_elu:
                o = jnp.where(o > 0, o, jnp.exp(jnp.minimum(o, 0.0)) - 1.0)
            o_ref[:, sc * 64:(sc + 1) * 64] = o

    bm = NP // 10
    return pl.pallas_call(
        body,
        grid=(10,),
        in_specs=[pl.BlockSpec((2, 2, bm, ACCW), lambda i: (0, 0, i, 0)),
                  pl.BlockSpec((bm, 8), lambda i: (i, 0)),
                  pl.BlockSpec((bm, 128), lambda i: (i, 0)),
                  pl.BlockSpec((1, 128), lambda i: (0, 0)),
                  pl.BlockSpec((4, 64), lambda i: (0, 0))],
        out_specs=pl.BlockSpec((bm, 128), lambda i: (i, 0)),
        out_shape=jax.ShapeDtypeStruct((NP, 128), jnp.float32),
    )(acc, exl, xp_sc, b, k4)


def _tc_pool(h3, p):
    def body(p_ref, h_ref, o_ref):
        pm = p_ref[...]
        s = lax.dot_general(pm, h_ref[...], (((0,), (0,)), ((), ())),
                            preferred_element_type=jnp.float32)
        cnt = jnp.sum(pm, axis=0)[:, None]
        o_ref[...] = s / jnp.maximum(cnt, 1.0)

    return pl.pallas_call(
        body,
        in_specs=[pl.BlockSpec((NP, G), lambda: (0, 0)),
                  pl.BlockSpec((NP, 128), lambda: (0, 0))],
        out_specs=pl.BlockSpec((G, 128), lambda: (0, 0)),
        out_shape=jax.ShapeDtypeStruct((G, 128), jnp.float32),
    )(p, h3)


def _mask8(a):
    return (jnp.eye(8, dtype=jnp.float32)[:, None, :] * a[0][:, :, None]).reshape(128, 8)


def _mask1(a):
    return jnp.tile(a[0, 0][:, None], (1, 8))


def kernel(x, edge_index, edge_attr, batch, W1, a_src1, a_dst1, We1, a_e1, b1,
           W2, a_src2, a_dst2, We2, a_e2, b2, W3, a_src3, a_dst3, We3, a_e3, b3):
    f32 = jnp.float32
    src = edge_index[0]
    dst = edge_index[1]
    epad = EPAD - E
    src_p = jnp.concatenate([src, jnp.full((epad,), N, jnp.int32)])
    dst_p = jnp.concatenate([dst, jnp.full((epad,), N, jnp.int32)])
    ea_p = jnp.concatenate([edge_attr, jnp.zeros((epad, EDIM), f32)])
    x_p = jnp.concatenate([x, jnp.zeros((NP - N, DIN), f32)])

    p = (batch[:, None] == jnp.arange(G, dtype=jnp.int32)[None, :]).astype(f32)
    p = jnp.concatenate([p, jnp.zeros((NP - N, G), f32)])

    k4 = (jnp.eye(4, dtype=f32)[:, :, None] * jnp.ones((1, 1, 16), f32)).reshape(4, 64)

    msrc = (_mask8(a_src1), _mask8(a_src2), _mask1(a_src3))
    mdst = (_mask8(a_dst1), _mask8(a_dst2), _mask1(a_dst3))
    me = (_mask8(a_e1), _mask8(a_e2), _mask1(a_e3))
    ws = (W1, W2, W3)
    wes = (We1, We2, We3)
    bs = (b1.reshape(1, 128), b2.reshape(1, 128), b3.reshape(1, 128))

    acca, accb = _sc_loop_attr(dst_p, ea_p)
    la = _tc_loop_finish(acca, accb)
    aes = _tc_edge_prep(ea_p, We1, me[0], We2, me[1], We3, me[2])

    h = x_p
    for l in range(3):
        xp, xps, asrc_t, adst_t, exl = _tc_prep(h, ws[l], msrc[l], mdst[l], la,
                                                wes[l], me[l])
        acc = _sc_edge(src_p, dst_p, aes[l], asrc_t, adst_t,
                       xps.reshape(4 * NP, 32))
        h = _tc_norm(acc.reshape(2, 2, NP, ACCW), exl, xp, bs[l], k4,
                     do_elu=(l < 2))

    out = _tc_pool(h, p)
    return out
